# baseline (speedup 1.0000x reference)
import functools

import jax
import jax.numpy as jnp
from jax import lax
from jax.experimental import pallas as pl
from jax.experimental.pallas import tpu as pltpu
from jax.experimental.pallas import tpu_sc as plsc

SEQ = 200
BATCH = 4096
D = 64
NW = 32
BPW = BATCH // NW
CB = 4
CHUNK = 100
CPS = CB * SEQ // CHUNK
NCHUNK = BPW * SEQ // CHUNK
STEPS = BPW // CB
NREG = D // 16

_mesh = plsc.VectorSubcoreMesh(core_axis_name="c", subcore_axis_name="s")


@functools.partial(
    pl.kernel,
    mesh=_mesh,
    out_type=jax.ShapeDtypeStruct((BATCH, D), jnp.float32),
    compiler_params=pltpu.CompilerParams(use_tc_tiling_on_sc=False),
    scratch_types=[
        pltpu.VMEM((NCHUNK, CHUNK), jnp.int32),
        pltpu.VMEM((CB * SEQ, D), jnp.float32),
        pltpu.VMEM((CB, D), jnp.float32),
        pltpu.SemaphoreType.DMA,
    ],
)
def _sum_embed(idx_hbm, tab_hbm, out_hbm, idx_v, rows_v, acc_v, sem):
    wid = lax.axis_index("s") * 2 + lax.axis_index("c")
    pltpu.sync_copy(idx_hbm.at[wid], idx_v)

    def step(i, carry):
        copies = [
            pltpu.async_copy(
                tab_hbm.at[idx_v.at[i * CPS + k]],
                rows_v.at[pl.ds(k * CHUNK, CHUNK)],
                sem,
            )
            for k in range(CPS)
        ]
        for cp in copies:
            cp.wait()

        def red(si, acc):
            out = []
            for b in range(CB):
                r = b * SEQ + si
                for k in range(NREG):
                    out.append(acc[b * NREG + k] + rows_v[r, pl.ds(k * 16, 16)])
            return tuple(out)

        acc = lax.fori_loop(
            0, SEQ, red,
            tuple(jnp.zeros((16,), jnp.float32) for _ in range(CB * NREG)),
        )
        for b in range(CB):
            for k in range(NREG):
                acc_v[b, pl.ds(k * 16, 16)] = acc[b * NREG + k]
        pltpu.sync_copy(acc_v, out_hbm.at[pl.ds(wid * BPW + i * CB, CB)])
        return carry

    lax.fori_loop(0, STEPS, step, 0)


def kernel(indices, table):
    idx3 = indices.T.reshape(NW, NCHUNK, CHUNK)
    return _sum_embed(idx3, table)


# speedup vs baseline: 1.0388x; 1.0388x over previous
import functools

import jax
import jax.numpy as jnp
from jax import lax
from jax.experimental import pallas as pl
from jax.experimental.pallas import tpu as pltpu
from jax.experimental.pallas import tpu_sc as plsc

SEQ = 200
BATCH = 4096
D = 64
NW = 32
BPW = BATCH // NW
NREG = D // 16
KH = 4
NGRP = SEQ // (2 * KH)
UB = 4

_mesh = plsc.VectorSubcoreMesh(core_axis_name="c", subcore_axis_name="s")


@functools.partial(
    pl.kernel,
    mesh=_mesh,
    out_type=jax.ShapeDtypeStruct((BATCH, D), jnp.float32),
    compiler_params=pltpu.CompilerParams(use_tc_tiling_on_sc=False),
    scratch_types=[
        pltpu.VMEM((SEQ, BPW), jnp.int32),
        pltpu.VMEM((2, KH, BPW, D), jnp.float32),
        pltpu.VMEM((BPW, D), jnp.float32),
        pltpu.SemaphoreType.DMA,
        pltpu.SemaphoreType.DMA,
    ],
)
def _sum_embed(idx_hbm, tab_hbm, out_hbm, idx_v, ring_v, acc_v, sem_a, sem_b):
    wid = lax.axis_index("s") * 2 + lax.axis_index("c")
    pltpu.sync_copy(idx_hbm.at[:, pl.ds(wid * BPW, BPW)], idx_v)

    def fire(s0, half, sem):
        for j in range(KH):
            pltpu.async_copy(
                tab_hbm.at[idx_v.at[s0 + j]], ring_v.at[half, j], sem)

    def drain(s0, half, sem):
        for j in range(KH):
            pltpu.make_async_copy(
                tab_hbm.at[idx_v.at[s0 + j]], ring_v.at[half, j], sem).wait()

    def reduce_half(half):
        for j in range(KH):
            def red(bi, c, j=j):
                for u in range(UB):
                    b = bi * UB + u
                    for k in range(NREG):
                        plsc.addupdate(
                            acc_v.at[b, pl.ds(k * 16, 16)],
                            ring_v[half, j, b, pl.ds(k * 16, 16)],
                        )
                return c
            lax.fori_loop(0, BPW // UB, red, 0)

    zvec = jnp.zeros((16,), jnp.float32)

    def zero(bi, c):
        for u in range(UB):
            for k in range(NREG):
                acc_v[bi * UB + u, pl.ds(k * 16, 16)] = zvec
        return c

    lax.fori_loop(0, BPW // UB, zero, 0)

    fire(0, 0, sem_a)

    def grp(g, carry):
        s0 = g * 2 * KH
        fire(s0 + KH, 1, sem_b)
        drain(s0, 0, sem_a)
        reduce_half(0)

        @pl.when(g < NGRP - 1)
        def _():
            fire(s0 + 2 * KH, 0, sem_a)

        drain(s0 + KH, 1, sem_b)
        reduce_half(1)
        return carry

    lax.fori_loop(0, NGRP, grp, 0)
    pltpu.sync_copy(acc_v, out_hbm.at[pl.ds(wid * BPW, BPW)])


def kernel(indices, table):
    return _sum_embed(indices, table)
